# baseline (speedup 1.0000x reference)
import functools

import jax
import jax.numpy as jnp
from jax import lax
from jax.experimental import pallas as pl
from jax.experimental.pallas import tpu as pltpu
from jax.experimental.pallas import tpu_sc as plsc

N_HEROES = 100000
EMB_DIM = 128
CONTEXT = 9
HIDDEN = 64

NW = 32
CPT = 1408
R_SC = NW * CPT
KSTAGE = 16
NSTAGE = HIDDEN // KSTAGE
BLKS = CPT // 16

NS = 4
CHUNK = 2048
PADN = 102400
TC_COLS = PADN - R_SC
K_TC = TC_COLS // (NS * CHUNK)
TC_BLK0 = R_SC // CHUNK
LAST_BLK = (N_HEROES - 1) // CHUNK


def _gather_fc1(ctx_ref, w1_ref, b1_ref, table_ref, emb_v, sem):
    for j in range(CONTEXT):
        pltpu.make_async_copy(
            table_ref.at[pl.ds(ctx_ref[j], 1), :],
            emb_v.at[pl.ds(j, 1), :], sem).start()
    for j in range(CONTEXT):
        pltpu.make_async_copy(
            table_ref.at[pl.ds(ctx_ref[j], 1), :],
            emb_v.at[pl.ds(j, 1), :], sem).wait()
    acc = jnp.zeros((1, HIDDEN), jnp.float32)
    for j in range(CONTEXT):
        acc += jnp.dot(
            emb_v[j : j + 1, :],
            w1_ref[EMB_DIM * j : EMB_DIM * (j + 1), :],
            preferred_element_type=jnp.float32,
        )
    return jnp.maximum(acc + jnp.reshape(b1_ref[...], (1, HIDDEN)), 0.0)


def _fc1_body(ctx_ref, w1_ref, b1_ref, table_ref, hs_ref, emb_v, sem):
    h = _gather_fc1(ctx_ref, w1_ref, b1_ref, table_ref, emb_v, sem)
    rows = lax.broadcasted_iota(jnp.int32, (HIDDEN, 16 * HIDDEN), 0)
    cols = lax.broadcasted_iota(jnp.int32, (HIDDEN, 16 * HIDDEN), 1)
    onehot = jnp.where(rows == cols // 16, 1.0, 0.0).astype(jnp.float32)
    hs = jnp.dot(h, onehot, preferred_element_type=jnp.float32)
    hs_ref[...] = jnp.reshape(hs, (16 * HIDDEN,))


def _fc1(context, emb_table, W1, b1):
    return pl.pallas_call(
        _fc1_body,
        in_specs=[
            pl.BlockSpec(memory_space=pltpu.MemorySpace.SMEM),
            pl.BlockSpec((CONTEXT * EMB_DIM, HIDDEN), lambda: (0, 0)),
            pl.BlockSpec((HIDDEN,), lambda: (0,)),
            pl.BlockSpec(memory_space=pltpu.MemorySpace.HBM),
        ],
        out_specs=pl.BlockSpec((16 * HIDDEN,), lambda: (0,)),
        out_shape=jax.ShapeDtypeStruct((16 * HIDDEN,), jnp.float32),
        scratch_shapes=[
            pltpu.VMEM((16, EMB_DIM), jnp.float32),
            pltpu.SemaphoreType.DMA,
        ],
    )(context, W1, b1, emb_table)


def _sc_vocab(hs, W2, b2):
    mesh = plsc.VectorSubcoreMesh(core_axis_name="c", subcore_axis_name="s")

    @functools.partial(
        pl.kernel,
        mesh=mesh,
        out_type=(
            jax.ShapeDtypeStruct((NW, CPT), jnp.float32),
            jax.ShapeDtypeStruct((NW, 16), jnp.float32),
            jax.ShapeDtypeStruct((NW, 16), jnp.float32),
        ),
        scratch_types=[
            pltpu.VMEM((16 * HIDDEN,), jnp.float32),
            pltpu.VMEM((HIDDEN * CPT,), jnp.float32),
            pltpu.VMEM((CPT,), jnp.float32),
            pltpu.VMEM((16,), jnp.float32),
            pltpu.VMEM((16,), jnp.float32),
            pltpu.SemaphoreType.DMA,
            pltpu.SemaphoreType.DMA,
            pltpu.SemaphoreType.DMA,
            pltpu.SemaphoreType.DMA,
            pltpu.SemaphoreType.DMA,
        ],
    )
    def sc_k(hs_hbm, w2_hbm, b2_hbm, lo_hbm, m_hbm, s_hbm,
             h_v, w_v, acc_v, m_v, s_v, sem0, sem1, sem2, sem3, semx):
        cid = lax.axis_index("c")
        sid = lax.axis_index("s")
        wid = sid * 2 + cid
        base = wid * CPT
        sems = (sem0, sem1, sem2, sem3)

        def stage_copies(st):
            for k in range(KSTAGE):
                row = st * KSTAGE + k
                yield pltpu.make_async_copy(
                    w2_hbm.at[row, pl.ds(base, CPT)],
                    w_v.at[pl.ds(row * CPT, CPT)], sems[st])

        pltpu.sync_copy(hs_hbm, h_v)
        pltpu.async_copy(b2_hbm.at[pl.ds(base, CPT)], acc_v, semx).start()
        for cp in stage_copies(0):
            cp.start()
        pltpu.make_async_copy(b2_hbm.at[pl.ds(base, CPT)], acc_v, semx).wait()
        for st in range(NSTAGE):
            if st + 1 < NSTAGE:
                for cp in stage_copies(st + 1):
                    cp.start()
            for cp in stage_copies(st):
                cp.wait()

            h0 = st * KSTAGE * 16
            w0 = st * KSTAGE * CPT

            def blkbody(blk, _):
                c0 = blk * 16
                v0 = h_v[pl.ds(h0, 16)] * w_v[pl.ds(w0 + c0, 16)]
                v1 = (h_v[pl.ds(h0 + 16, 16)]
                      * w_v[pl.ds(w0 + CPT + c0, 16)])
                v2 = (h_v[pl.ds(h0 + 32, 16)]
                      * w_v[pl.ds(w0 + 2 * CPT + c0, 16)])
                v3 = (h_v[pl.ds(h0 + 48, 16)]
                      * w_v[pl.ds(w0 + 3 * CPT + c0, 16)])
                for k in range(4, KSTAGE, 4):
                    v0 = v0 + (h_v[pl.ds(h0 + k * 16, 16)]
                               * w_v[pl.ds(w0 + k * CPT + c0, 16)])
                    v1 = v1 + (h_v[pl.ds(h0 + (k + 1) * 16, 16)]
                               * w_v[pl.ds(w0 + (k + 1) * CPT + c0, 16)])
                    v2 = v2 + (h_v[pl.ds(h0 + (k + 2) * 16, 16)]
                               * w_v[pl.ds(w0 + (k + 2) * CPT + c0, 16)])
                    v3 = v3 + (h_v[pl.ds(h0 + (k + 3) * 16, 16)]
                               * w_v[pl.ds(w0 + (k + 3) * CPT + c0, 16)])
                plsc.addupdate(acc_v.at[pl.ds(c0, 16)],
                               (v0 + v1) + (v2 + v3))
                return 0

            lax.fori_loop(0, BLKS, blkbody, 0)

        mv = acc_v[pl.ds(0, 16)]
        for blk in range(1, BLKS):
            mv = jnp.maximum(mv, acc_v[pl.ds(blk * 16, 16)])
        sv = jnp.zeros((16,), jnp.float32)
        for blk in range(BLKS):
            sv = sv + jnp.exp(acc_v[pl.ds(blk * 16, 16)] - mv)
        m_v[...] = mv
        s_v[...] = sv
        pltpu.sync_copy(acc_v, lo_hbm.at[wid])
        pltpu.sync_copy(m_v, m_hbm.at[wid])
        pltpu.sync_copy(s_v, s_hbm.at[wid])

    return sc_k(hs, W2, b2)


def _tc_body(ctx_ref, w1_ref, b1_ref, table_ref, *refs):
    w2_refs = refs[:NS]
    b2_refs = refs[NS : 2 * NS]
    sc_ref, mp_ref, sp_ref = refs[2 * NS : 2 * NS + 3]
    out_ref = refs[2 * NS + 3]
    lt_v, h_v, emb_v, sem, m_s, s_s = refs[2 * NS + 4 :]
    i = pl.program_id(0)

    @pl.when(i == 0)
    def _():
        h_v[...] = _gather_fc1(ctx_ref, w1_ref, b1_ref, table_ref,
                               emb_v, sem)
        m_s[0, 0] = -1e30
        s_s[0, 0] = 0.0

    h = h_v[...]
    iota = lax.broadcasted_iota(jnp.int32, (1, CHUNK), 1)
    m_c = jnp.float32(-1e30)
    ls = []
    for s in range(NS):
        l = (jnp.dot(h, w2_refs[s][...], preferred_element_type=jnp.float32)
             + jnp.reshape(b2_refs[s][...], (1, CHUNK)))
        lt_v[0:1, pl.ds((NS * i + s) * CHUNK, CHUNK)] = l
        col0 = R_SC + (NS * i + s) * CHUNK
        l_m = jnp.where(col0 + iota < N_HEROES, l, -1e30)
        ls.append(l_m)
        m_c = jnp.maximum(m_c, jnp.max(l_m))
    s_c = jnp.float32(0.0)
    for l_m in ls:
        s_c = s_c + jnp.sum(jnp.exp(l_m - m_c))
    m_old = m_s[0, 0]
    s_old = s_s[0, 0]
    m_new = jnp.maximum(m_old, m_c)
    s_new = s_old * jnp.exp(m_old - m_new) + s_c * jnp.exp(m_c - m_new)
    m_s[0, 0] = m_new
    s_s[0, 0] = s_new

    @pl.when(i == K_TC - 1)
    def _():
        mp = mp_ref[...]
        sp = sp_ref[...]
        m_star = jnp.maximum(m_new, jnp.max(mp))
        s_star = (s_new * jnp.exp(m_new - m_star)
                  + jnp.sum(sp * jnp.exp(mp - m_star)))
        log_z = m_star + jnp.log(s_star)
        for wid in range(NW):
            out_ref[0:1, wid * CPT : (wid + 1) * CPT] = (
                sc_ref[wid : wid + 1, :] - log_z)
        out_ref[0:1, R_SC:N_HEROES] = (
            lt_v[0:1, 0 : N_HEROES - R_SC] - log_z)


def _mk_blk(s, one_d):
    if one_d:
        return lambda i: (jnp.minimum(TC_BLK0 + NS * i + s, LAST_BLK),)
    return lambda i: (0, jnp.minimum(TC_BLK0 + NS * i + s, LAST_BLK))


def _tc_vocab(context, emb_table, W1, b1, W2, b2, lo_sc, mp, sp):
    return pl.pallas_call(
        _tc_body,
        grid=(K_TC,),
        in_specs=[
            pl.BlockSpec(memory_space=pltpu.MemorySpace.SMEM),
            pl.BlockSpec((CONTEXT * EMB_DIM, HIDDEN), lambda i: (0, 0)),
            pl.BlockSpec((HIDDEN,), lambda i: (0,)),
            pl.BlockSpec(memory_space=pltpu.MemorySpace.HBM),
        ]
        + [pl.BlockSpec((HIDDEN, CHUNK), _mk_blk(s, False))
           for s in range(NS)]
        + [pl.BlockSpec((CHUNK,), _mk_blk(s, True)) for s in range(NS)]
        + [
            pl.BlockSpec((NW, CPT), lambda i: (0, 0)),
            pl.BlockSpec((NW, 16), lambda i: (0, 0)),
            pl.BlockSpec((NW, 16), lambda i: (0, 0)),
        ],
        out_specs=pl.BlockSpec((1, N_HEROES), lambda i: (0, 0)),
        out_shape=jax.ShapeDtypeStruct((1, N_HEROES), jnp.float32),
        scratch_shapes=[
            pltpu.VMEM((1, TC_COLS), jnp.float32),
            pltpu.VMEM((1, HIDDEN), jnp.float32),
            pltpu.VMEM((16, EMB_DIM), jnp.float32),
            pltpu.SemaphoreType.DMA,
            pltpu.SMEM((1, 1), jnp.float32),
            pltpu.SMEM((1, 1), jnp.float32),
        ],
    )(context, W1, b1, emb_table, *([W2] * NS), *([b2] * NS),
      lo_sc, mp, sp)


def kernel(context, emb_table, W1, b1, W2, b2):
    hs = _fc1(context, emb_table, W1, b1)
    lo_sc, mp, sp = _sc_vocab(hs, W2, b2)
    return _tc_vocab(context, emb_table, W1, b1, W2, b2, lo_sc, mp, sp)


# speedup vs baseline: 1.1955x; 1.1955x over previous
import functools

import jax
import jax.numpy as jnp
from jax import lax
from jax.experimental import pallas as pl
from jax.experimental.pallas import tpu as pltpu
from jax.experimental.pallas import tpu_sc as plsc

N_HEROES = 100000
EMB_DIM = 128
CONTEXT = 9
HIDDEN = 64

NW = 32
CPT = 1408
R_SC = NW * CPT
KSTAGE = 16
NSTAGE = HIDDEN // KSTAGE
BLKS = CPT // 16

NS = 4
CHUNK = 2048
PADN = 102400
TC_COLS = PADN - R_SC
K_TC = TC_COLS // (NS * CHUNK)
TC_BLK0 = R_SC // CHUNK
LAST_BLK = (N_HEROES - 1) // CHUNK


def _gather_fc1(ctx_ref, w1_ref, b1_ref, table_ref, emb_v, sem):
    for j in range(CONTEXT):
        pltpu.make_async_copy(
            table_ref.at[pl.ds(ctx_ref[j], 1), :],
            emb_v.at[pl.ds(j, 1), :], sem).start()
    for j in range(CONTEXT):
        pltpu.make_async_copy(
            table_ref.at[pl.ds(ctx_ref[j], 1), :],
            emb_v.at[pl.ds(j, 1), :], sem).wait()
    acc = jnp.zeros((1, HIDDEN), jnp.float32)
    for j in range(CONTEXT):
        acc += jnp.dot(
            emb_v[j : j + 1, :],
            w1_ref[EMB_DIM * j : EMB_DIM * (j + 1), :],
            preferred_element_type=jnp.float32,
        )
    return jnp.maximum(acc + jnp.reshape(b1_ref[...], (1, HIDDEN)), 0.0)


def _fc1_body(ctx_ref, w1_ref, b1_ref, table_ref, hs_ref, emb_v, sem):
    h = _gather_fc1(ctx_ref, w1_ref, b1_ref, table_ref, emb_v, sem)
    rows = lax.broadcasted_iota(jnp.int32, (HIDDEN, 16 * HIDDEN), 0)
    cols = lax.broadcasted_iota(jnp.int32, (HIDDEN, 16 * HIDDEN), 1)
    onehot = jnp.where(rows == cols // 16, 1.0, 0.0).astype(jnp.float32)
    hs = jnp.dot(h, onehot, preferred_element_type=jnp.float32)
    hs_ref[...] = jnp.reshape(hs, (16 * HIDDEN,))


def _fc1(context, emb_table, W1, b1):
    return pl.pallas_call(
        _fc1_body,
        in_specs=[
            pl.BlockSpec(memory_space=pltpu.MemorySpace.SMEM),
            pl.BlockSpec((CONTEXT * EMB_DIM, HIDDEN), lambda: (0, 0)),
            pl.BlockSpec((HIDDEN,), lambda: (0,)),
            pl.BlockSpec(memory_space=pltpu.MemorySpace.HBM),
        ],
        out_specs=pl.BlockSpec((16 * HIDDEN,), lambda: (0,)),
        out_shape=jax.ShapeDtypeStruct((16 * HIDDEN,), jnp.float32),
        scratch_shapes=[
            pltpu.VMEM((16, EMB_DIM), jnp.float32),
            pltpu.SemaphoreType.DMA,
        ],
    )(context, W1, b1, emb_table)


def _sc_vocab(hs, W2, b2):
    mesh = plsc.VectorSubcoreMesh(core_axis_name="c", subcore_axis_name="s")

    @functools.partial(
        pl.kernel,
        mesh=mesh,
        out_type=(
            jax.ShapeDtypeStruct((NW, CPT), jnp.float32),
            jax.ShapeDtypeStruct((NW, 16), jnp.float32),
            jax.ShapeDtypeStruct((NW, 16), jnp.float32),
        ),
        scratch_types=[
            pltpu.VMEM((16 * HIDDEN,), jnp.float32),
            pltpu.VMEM((HIDDEN * CPT,), jnp.float32),
            pltpu.VMEM((CPT,), jnp.float32),
            pltpu.VMEM((16,), jnp.float32),
            pltpu.VMEM((16,), jnp.float32),
            pltpu.SemaphoreType.DMA,
            pltpu.SemaphoreType.DMA,
            pltpu.SemaphoreType.DMA,
            pltpu.SemaphoreType.DMA,
            pltpu.SemaphoreType.DMA,
        ],
    )
    def sc_k(hs_hbm, w2_hbm, b2_hbm, lo_hbm, m_hbm, s_hbm,
             h_v, w_v, acc_v, m_v, s_v, sem0, sem1, sem2, sem3, semx):
        cid = lax.axis_index("c")
        sid = lax.axis_index("s")
        wid = sid * 2 + cid
        base = wid * CPT
        sems = (sem0, sem1, sem2, sem3)

        def stage_copies(st):
            for k in range(KSTAGE):
                row = st * KSTAGE + k
                yield pltpu.make_async_copy(
                    w2_hbm.at[row, pl.ds(base, CPT)],
                    w_v.at[pl.ds(row * CPT, CPT)], sems[st])

        pltpu.sync_copy(hs_hbm, h_v)
        pltpu.async_copy(b2_hbm.at[pl.ds(base, CPT)], acc_v, semx).start()
        for cp in stage_copies(0):
            cp.start()
        pltpu.make_async_copy(b2_hbm.at[pl.ds(base, CPT)], acc_v, semx).wait()
        for st in range(NSTAGE):
            if st + 1 < NSTAGE:
                for cp in stage_copies(st + 1):
                    cp.start()
            for cp in stage_copies(st):
                cp.wait()

            h0 = st * KSTAGE * 16
            w0 = st * KSTAGE * CPT

            def blkbody(blk, _):
                c0 = blk * 16
                v0 = h_v[pl.ds(h0, 16)] * w_v[pl.ds(w0 + c0, 16)]
                v1 = (h_v[pl.ds(h0 + 16, 16)]
                      * w_v[pl.ds(w0 + CPT + c0, 16)])
                v2 = (h_v[pl.ds(h0 + 32, 16)]
                      * w_v[pl.ds(w0 + 2 * CPT + c0, 16)])
                v3 = (h_v[pl.ds(h0 + 48, 16)]
                      * w_v[pl.ds(w0 + 3 * CPT + c0, 16)])
                for k in range(4, KSTAGE, 4):
                    v0 = v0 + (h_v[pl.ds(h0 + k * 16, 16)]
                               * w_v[pl.ds(w0 + k * CPT + c0, 16)])
                    v1 = v1 + (h_v[pl.ds(h0 + (k + 1) * 16, 16)]
                               * w_v[pl.ds(w0 + (k + 1) * CPT + c0, 16)])
                    v2 = v2 + (h_v[pl.ds(h0 + (k + 2) * 16, 16)]
                               * w_v[pl.ds(w0 + (k + 2) * CPT + c0, 16)])
                    v3 = v3 + (h_v[pl.ds(h0 + (k + 3) * 16, 16)]
                               * w_v[pl.ds(w0 + (k + 3) * CPT + c0, 16)])
                plsc.addupdate(acc_v.at[pl.ds(c0, 16)],
                               (v0 + v1) + (v2 + v3))
                return 0

            lax.fori_loop(0, BLKS, blkbody, 0)

        mv = acc_v[pl.ds(0, 16)]
        for blk in range(1, BLKS):
            mv = jnp.maximum(mv, acc_v[pl.ds(blk * 16, 16)])
        sv = jnp.zeros((16,), jnp.float32)
        for blk in range(BLKS):
            sv = sv + jnp.exp(acc_v[pl.ds(blk * 16, 16)] - mv)
        m_v[...] = mv
        s_v[...] = sv
        pltpu.sync_copy(acc_v, lo_hbm.at[wid])
        pltpu.sync_copy(m_v, m_hbm.at[wid])
        pltpu.sync_copy(s_v, s_hbm.at[wid])

    return sc_k(hs, W2, b2)


def _tc_body(ctx_ref, w1_ref, b1_ref, table_ref, *refs):
    w2_refs = refs[:NS]
    b2_refs = refs[NS : 2 * NS]
    out_ref, mt_ref, st_ref = refs[2 * NS : 2 * NS + 3]
    h_v, emb_v, sem, m_s, s_s = refs[2 * NS + 3 :]
    i = pl.program_id(0)

    @pl.when(i == 0)
    def _():
        h_v[...] = _gather_fc1(ctx_ref, w1_ref, b1_ref, table_ref,
                               emb_v, sem)
        m_s[0, 0] = -1e30
        s_s[0, 0] = 0.0

    h = h_v[...]
    iota = lax.broadcasted_iota(jnp.int32, (1, CHUNK), 1)
    m_c = jnp.float32(-1e30)
    ls = []
    for s in range(NS):
        l = (jnp.dot(h, w2_refs[s][...], preferred_element_type=jnp.float32)
             + jnp.reshape(b2_refs[s][...], (1, CHUNK)))
        out_ref[0:1, s * CHUNK : (s + 1) * CHUNK] = l
        col0 = R_SC + (NS * i + s) * CHUNK
        l_m = jnp.where(col0 + iota < N_HEROES, l, -1e30)
        ls.append(l_m)
        m_c = jnp.maximum(m_c, jnp.max(l_m))
    s_c = jnp.float32(0.0)
    for l_m in ls:
        s_c = s_c + jnp.sum(jnp.exp(l_m - m_c))
    m_old = m_s[0, 0]
    s_old = s_s[0, 0]
    m_new = jnp.maximum(m_old, m_c)
    s_new = s_old * jnp.exp(m_old - m_new) + s_c * jnp.exp(m_c - m_new)
    m_s[0, 0] = m_new
    s_s[0, 0] = s_new

    @pl.when(i == K_TC - 1)
    def _():
        mt_ref[...] = jnp.full((1, 128), m_new, jnp.float32)
        st_ref[...] = jnp.full((1, 128), s_new, jnp.float32)


def _mk_blk(s, one_d):
    if one_d:
        return lambda i: (jnp.minimum(TC_BLK0 + NS * i + s, LAST_BLK),)
    return lambda i: (0, jnp.minimum(TC_BLK0 + NS * i + s, LAST_BLK))


def _tc_vocab(context, emb_table, W1, b1, W2, b2):
    return pl.pallas_call(
        _tc_body,
        grid=(K_TC,),
        in_specs=[
            pl.BlockSpec(memory_space=pltpu.MemorySpace.SMEM),
            pl.BlockSpec((CONTEXT * EMB_DIM, HIDDEN), lambda i: (0, 0)),
            pl.BlockSpec((HIDDEN,), lambda i: (0,)),
            pl.BlockSpec(memory_space=pltpu.MemorySpace.HBM),
        ]
        + [pl.BlockSpec((HIDDEN, CHUNK), _mk_blk(s, False))
           for s in range(NS)]
        + [pl.BlockSpec((CHUNK,), _mk_blk(s, True)) for s in range(NS)],
        out_specs=(
            pl.BlockSpec((1, NS * CHUNK), lambda i: (0, i)),
            pl.BlockSpec((1, 128), lambda i: (0, 0)),
            pl.BlockSpec((1, 128), lambda i: (0, 0)),
        ),
        out_shape=(
            jax.ShapeDtypeStruct((1, TC_COLS), jnp.float32),
            jax.ShapeDtypeStruct((1, 128), jnp.float32),
            jax.ShapeDtypeStruct((1, 128), jnp.float32),
        ),
        scratch_shapes=[
            pltpu.VMEM((1, HIDDEN), jnp.float32),
            pltpu.VMEM((16, EMB_DIM), jnp.float32),
            pltpu.SemaphoreType.DMA,
            pltpu.SMEM((1, 1), jnp.float32),
            pltpu.SMEM((1, 1), jnp.float32),
        ],
    )(context, W1, b1, emb_table, *([W2] * NS), *([b2] * NS))


def _norm_body(tc_ref, sc_ref, mt_ref, st_ref, mp_ref, sp_ref, out_ref):
    m_tc = mt_ref[0, 0]
    s_tc = st_ref[0, 0]
    mp = mp_ref[...]
    sp = sp_ref[...]
    m_star = jnp.maximum(m_tc, jnp.max(mp))
    s_star = s_tc * jnp.exp(m_tc - m_star) + jnp.sum(sp * jnp.exp(mp - m_star))
    log_z = m_star + jnp.log(s_star)
    for wid in range(NW):
        out_ref[0:1, wid * CPT : (wid + 1) * CPT] = (
            sc_ref[wid : wid + 1, :] - log_z)
    out_ref[0:1, R_SC:N_HEROES] = tc_ref[0:1, 0 : N_HEROES - R_SC] - log_z


def _normalize(lo_tc, lo_sc, mt, st, mp, sp):
    return pl.pallas_call(
        _norm_body,
        in_specs=[
            pl.BlockSpec((1, TC_COLS), lambda: (0, 0)),
            pl.BlockSpec((NW, CPT), lambda: (0, 0)),
            pl.BlockSpec((1, 128), lambda: (0, 0)),
            pl.BlockSpec((1, 128), lambda: (0, 0)),
            pl.BlockSpec((NW, 16), lambda: (0, 0)),
            pl.BlockSpec((NW, 16), lambda: (0, 0)),
        ],
        out_specs=pl.BlockSpec((1, N_HEROES), lambda: (0, 0)),
        out_shape=jax.ShapeDtypeStruct((1, N_HEROES), jnp.float32),
    )(lo_tc, lo_sc, mt, st, mp, sp)


def kernel(context, emb_table, W1, b1, W2, b2):
    hs = _fc1(context, emb_table, W1, b1)
    lo_sc, mp, sp = _sc_vocab(hs, W2, b2)
    lo_tc, mt, st = _tc_vocab(context, emb_table, W1, b1, W2, b2)
    return _normalize(lo_tc, lo_sc, mt, st, mp, sp)
